# baseline (speedup 1.0000x reference)
import functools

import jax
import jax.numpy as jnp
from jax import lax
from jax.experimental import pallas as pl
from jax.experimental.pallas import tpu as pltpu
from jax.experimental.pallas import tpu_sc as plsc

_REPLACE_ID = 100001

_B = 128
_L = 2048
_LANES = 16
_NC = 2
_NS = 16
_NW = _NC * _NS
_ROWS_PER_W = _B // _NW
_PAIR = 2
_BANKS = 8


def _prep_body(pml_ref, p_ref, c_ref):
    x = pml_ref[...]
    p_ref[...] = 1.0 / (1.0 + jnp.exp(-x))
    sp = jnp.maximum(x, 0.0) + jnp.log1p(jnp.exp(-jnp.abs(x)))
    c_ref[...] = jnp.broadcast_to(jnp.sum(sp), (_LANES,))


def _sc_body(seq_hbm, u_hbm, p_hbm, pml_hbm, c_hbm,
             seq_out_hbm, mask_hbm, logits_hbm,
             p_v, pml_v, c_v,
             seq0_v, u0_v, so0_v, mk0_v,
             seq1_v, u1_v, so1_v, mk1_v,
             logits_v, acc_v,
             sem_m, sem_0, sem_1, sem_o):
    cid = lax.axis_index("c")
    sid = lax.axis_index("s")
    wid = cid * _NS + sid
    base = wid * _ROWS_PER_W

    cp = pltpu.async_copy
    d_p = cp(p_hbm, p_v, sem_m)
    d_l = cp(pml_hbm, pml_v, sem_m)
    d_c = cp(c_hbm, c_v, sem_m)
    d_s0 = cp(seq_hbm.at[pl.ds(base, _PAIR)], seq0_v, sem_0)
    d_u0 = cp(u_hbm.at[pl.ds(base, _PAIR)], u0_v, sem_0)
    d_s1 = cp(seq_hbm.at[pl.ds(base + _PAIR, _PAIR)], seq1_v, sem_1)
    d_u1 = cp(u_hbm.at[pl.ds(base + _PAIR, _PAIR)], u1_v, sem_1)

    zero = jnp.zeros((_LANES,), jnp.float32)

    def run_pair(seq_v, u_v, so_v, mk_v, acc_v):
        for b in range(_PAIR * _BANKS):
            acc_v[b, :] = zero

        @plsc.parallel_loop(0, _L, step=_LANES, unroll=_BANKS)
        def _(off):
            sl = pl.ds(off, _LANES)
            bank = (off // _LANES) % _BANKS
            pv = p_v[sl]
            lv = pml_v[sl]
            for r in range(_PAIR):
                m = u_v[r, sl] < pv
                mk_v[r, sl] = jnp.where(m, 1.0, 0.0).astype(jnp.float32)
                so_v[r, sl] = jnp.where(m, _REPLACE_ID, seq_v[r, sl])
                plsc.addupdate(acc_v.at[r * _BANKS + bank],
                               jnp.where(m, lv, 0.0))

        out = []
        for r in range(_PAIR):
            t = acc_v[r * _BANKS, :]
            for b in range(1, _BANKS):
                t = t + acc_v[r * _BANKS + b, :]
            out.append(t)
        return out

    d_p.wait()
    d_l.wait()
    d_s0.wait()
    d_u0.wait()
    accs0 = run_pair(seq0_v, u0_v, so0_v, mk0_v, acc_v)
    o_s0 = cp(so0_v, seq_out_hbm.at[pl.ds(base, _PAIR)], sem_o)
    o_m0 = cp(mk0_v, mask_hbm.at[pl.ds(base, _PAIR)], sem_o)

    d_s1.wait()
    d_u1.wait()
    accs1 = run_pair(seq1_v, u1_v, so1_v, mk1_v, acc_v)
    o_s1 = cp(so1_v, seq_out_hbm.at[pl.ds(base + _PAIR, _PAIR)], sem_o)
    o_m1 = cp(mk1_v, mask_hbm.at[pl.ds(base + _PAIR, _PAIR)], sem_o)

    d_c.wait()
    cvec = c_v[:]
    for r in range(_PAIR):
        logits_v[r, :] = jnp.broadcast_to(jnp.sum(accs0[r]), (_LANES,)) - cvec
        logits_v[_PAIR + r, :] = (
            jnp.broadcast_to(jnp.sum(accs1[r]), (_LANES,)) - cvec)

    pltpu.sync_copy(logits_v, logits_hbm.at[pl.ds(base, _ROWS_PER_W)])

    o_s0.wait()
    o_m0.wait()
    o_s1.wait()
    o_m1.wait()


@jax.jit
def kernel(sequence, prob_mask_logits, u):
    B, L = sequence.shape

    p2, c = pl.pallas_call(
        _prep_body,
        out_shape=(
            jax.ShapeDtypeStruct((_LANES, L // _LANES), jnp.float32),
            jax.ShapeDtypeStruct((_LANES,), jnp.float32),
        ),
    )(prob_mask_logits.reshape(_LANES, L // _LANES))
    p = p2.reshape(L)

    mesh = plsc.VectorSubcoreMesh(
        core_axis_name="c", subcore_axis_name="s",
        num_cores=_NC, num_subcores=_NS)

    sc = pl.kernel(
        _sc_body,
        out_type=(
            jax.ShapeDtypeStruct((B, L), jnp.int32),
            jax.ShapeDtypeStruct((B, L), jnp.float32),
            jax.ShapeDtypeStruct((B, _LANES), jnp.float32),
        ),
        mesh=mesh,
        compiler_params=pltpu.CompilerParams(needs_layout_passes=False),
        scratch_types=[
            pltpu.VMEM((L,), jnp.float32),
            pltpu.VMEM((L,), jnp.float32),
            pltpu.VMEM((_LANES,), jnp.float32),
            pltpu.VMEM((_PAIR, L), jnp.int32),
            pltpu.VMEM((_PAIR, L), jnp.float32),
            pltpu.VMEM((_PAIR, L), jnp.int32),
            pltpu.VMEM((_PAIR, L), jnp.float32),
            pltpu.VMEM((_PAIR, L), jnp.int32),
            pltpu.VMEM((_PAIR, L), jnp.float32),
            pltpu.VMEM((_PAIR, L), jnp.int32),
            pltpu.VMEM((_PAIR, L), jnp.float32),
            pltpu.VMEM((_ROWS_PER_W, _LANES), jnp.float32),
            pltpu.VMEM((_PAIR * _BANKS, _LANES), jnp.float32),
            pltpu.SemaphoreType.DMA,
            pltpu.SemaphoreType.DMA,
            pltpu.SemaphoreType.DMA,
            pltpu.SemaphoreType.DMA,
        ],
    )
    seq_out, hard_mask, logits_pad = sc(sequence, u, p, prob_mask_logits, c)
    return (seq_out, logits_pad[:, 0], hard_mask)


# speedup vs baseline: 1.0312x; 1.0312x over previous
import functools

import jax
import jax.numpy as jnp
from jax import lax
from jax.experimental import pallas as pl
from jax.experimental.pallas import tpu as pltpu
from jax.experimental.pallas import tpu_sc as plsc

_REPLACE_ID = 100001

_B = 128
_L = 2048
_LANES = 16
_NC = 2
_NS = 16
_NW = _NC * _NS
_ROWS_PER_W = _B // _NW
_PAIR = 2
_BANKS = 8


def _prep_body(pml_ref, pc_ref):
    x = pml_ref[...]
    pc_ref[0] = 1.0 / (1.0 + jnp.exp(-x))
    pc_ref[1] = x
    sp = jnp.maximum(x, 0.0) + jnp.log1p(jnp.exp(-jnp.abs(x)))
    pc_ref[2] = jnp.broadcast_to(jnp.sum(sp), x.shape)


def _sc_body(seq_hbm, u_hbm, pc_hbm,
             seq_out_hbm, mask_hbm, logits_hbm,
             pc_v,
             seq0_v, u0_v, so0_v, mk0_v,
             seq1_v, u1_v, so1_v, mk1_v,
             logits_v,
             sem_m, sem_0, sem_1, sem_o):
    cid = lax.axis_index("c")
    sid = lax.axis_index("s")
    wid = cid * _NS + sid
    base = wid * _ROWS_PER_W

    cp = pltpu.async_copy
    d_pc = cp(pc_hbm, pc_v, sem_m)
    d_s0 = cp(seq_hbm.at[pl.ds(base, _PAIR)], seq0_v, sem_0)
    d_u0 = cp(u_hbm.at[pl.ds(base, _PAIR)], u0_v, sem_0)
    d_s1 = cp(seq_hbm.at[pl.ds(base + _PAIR, _PAIR)], seq1_v, sem_1)
    d_u1 = cp(u_hbm.at[pl.ds(base + _PAIR, _PAIR)], u1_v, sem_1)

    zero = jnp.zeros((_LANES,), jnp.float32)

    def run_pair(seq_v, u_v, so_v, mk_v):
        @plsc.parallel_loop(0, _L, step=_LANES, unroll=4,
                            carry=(zero,) * _PAIR)
        def accs(off, carry):
            sl = pl.ds(off, _LANES)
            pv = pc_v[0, sl]
            lv = pc_v[1, sl]
            out = []
            for r in range(_PAIR):
                m = u_v[r, sl] < pv
                mk_v[r, sl] = jnp.where(m, 1.0, 0.0).astype(jnp.float32)
                so_v[r, sl] = jnp.where(m, _REPLACE_ID, seq_v[r, sl])
                out.append(carry[r] + jnp.where(m, lv, 0.0))
            return tuple(out)
        return accs

    d_pc.wait()
    d_s0.wait()
    d_u0.wait()
    accs0 = run_pair(seq0_v, u0_v, so0_v, mk0_v)
    o_s0 = cp(so0_v, seq_out_hbm.at[pl.ds(base, _PAIR)], sem_o)
    o_m0 = cp(mk0_v, mask_hbm.at[pl.ds(base, _PAIR)], sem_o)

    d_s1.wait()
    d_u1.wait()
    accs1 = run_pair(seq1_v, u1_v, so1_v, mk1_v)
    o_s1 = cp(so1_v, seq_out_hbm.at[pl.ds(base + _PAIR, _PAIR)], sem_o)
    o_m1 = cp(mk1_v, mask_hbm.at[pl.ds(base + _PAIR, _PAIR)], sem_o)

    cvec = pc_v[2, pl.ds(0, _LANES)]
    for r in range(_PAIR):
        logits_v[r, :] = jnp.broadcast_to(jnp.sum(accs0[r]), (_LANES,)) - cvec
        logits_v[_PAIR + r, :] = (
            jnp.broadcast_to(jnp.sum(accs1[r]), (_LANES,)) - cvec)

    pltpu.sync_copy(logits_v, logits_hbm.at[pl.ds(base, _ROWS_PER_W)])

    o_s0.wait()
    o_m0.wait()
    o_s1.wait()
    o_m1.wait()


@jax.jit
def kernel(sequence, prob_mask_logits, u):
    B, L = sequence.shape

    pc3 = pl.pallas_call(
        _prep_body,
        out_shape=jax.ShapeDtypeStruct((3, _LANES, L // _LANES), jnp.float32),
    )(prob_mask_logits.reshape(_LANES, L // _LANES))
    pc = pc3.reshape(3, L)

    mesh = plsc.VectorSubcoreMesh(
        core_axis_name="c", subcore_axis_name="s",
        num_cores=_NC, num_subcores=_NS)

    sc = pl.kernel(
        _sc_body,
        out_type=(
            jax.ShapeDtypeStruct((B, L), jnp.int32),
            jax.ShapeDtypeStruct((B, L), jnp.float32),
            jax.ShapeDtypeStruct((B, _LANES), jnp.float32),
        ),
        mesh=mesh,
        compiler_params=pltpu.CompilerParams(needs_layout_passes=False),
        scratch_types=[
            pltpu.VMEM((3, L), jnp.float32),
            pltpu.VMEM((_PAIR, L), jnp.int32),
            pltpu.VMEM((_PAIR, L), jnp.float32),
            pltpu.VMEM((_PAIR, L), jnp.int32),
            pltpu.VMEM((_PAIR, L), jnp.float32),
            pltpu.VMEM((_PAIR, L), jnp.int32),
            pltpu.VMEM((_PAIR, L), jnp.float32),
            pltpu.VMEM((_PAIR, L), jnp.int32),
            pltpu.VMEM((_PAIR, L), jnp.float32),
            pltpu.VMEM((_ROWS_PER_W, _LANES), jnp.float32),
            pltpu.SemaphoreType.DMA,
            pltpu.SemaphoreType.DMA,
            pltpu.SemaphoreType.DMA,
            pltpu.SemaphoreType.DMA,
        ],
    )
    seq_out, hard_mask, logits_pad = sc(sequence, u, pc)
    return (seq_out, logits_pad[:, 0], hard_mask)


# speedup vs baseline: 1.0321x; 1.0009x over previous
import functools

import jax
import jax.numpy as jnp
from jax import lax
from jax.experimental import pallas as pl
from jax.experimental.pallas import tpu as pltpu
from jax.experimental.pallas import tpu_sc as plsc

_REPLACE_ID = 100001

_B = 128
_L = 2048
_LANES = 16
_NC = 2
_NS = 16
_NW = _NC * _NS
_ROWS_PER_W = _B // _NW
_PAIR = 2
_BANKS = 8


def _prep_body(pml_ref, pc_ref, c_ref):
    x = pml_ref[...]
    pc_ref[0] = 1.0 / (1.0 + jnp.exp(-x))
    pc_ref[1] = x
    sp = jnp.maximum(x, 0.0) + jnp.log1p(jnp.exp(-jnp.abs(x)))
    c_ref[...] = jnp.broadcast_to(jnp.sum(sp), (_LANES,))


def _sc_body(seq_hbm, u_hbm, pc_hbm, c_hbm,
             seq_out_hbm, mask_hbm, logits_hbm,
             pc_v, c_v,
             seq0_v, u0_v, so0_v, mk0_v,
             seq1_v, u1_v, so1_v, mk1_v,
             logits_v,
             sem_m, sem_0, sem_1, sem_o):
    cid = lax.axis_index("c")
    sid = lax.axis_index("s")
    wid = cid * _NS + sid
    base = wid * _ROWS_PER_W

    cp = pltpu.async_copy
    d_pc = cp(pc_hbm, pc_v, sem_m)
    d_c = cp(c_hbm, c_v, sem_m)
    d_s0 = cp(seq_hbm.at[pl.ds(base, _PAIR)], seq0_v, sem_0)
    d_u0 = cp(u_hbm.at[pl.ds(base, _PAIR)], u0_v, sem_0)
    d_s1 = cp(seq_hbm.at[pl.ds(base + _PAIR, _PAIR)], seq1_v, sem_1)
    d_u1 = cp(u_hbm.at[pl.ds(base + _PAIR, _PAIR)], u1_v, sem_1)

    zero = jnp.zeros((_LANES,), jnp.float32)

    def run_pair(seq_v, u_v, so_v, mk_v):
        @plsc.parallel_loop(0, _L, step=_LANES, unroll=4,
                            carry=(zero,) * _PAIR)
        def accs(off, carry):
            sl = pl.ds(off, _LANES)
            pv = pc_v[0, sl]
            lv = pc_v[1, sl]
            out = []
            for r in range(_PAIR):
                m = u_v[r, sl] < pv
                mk_v[r, sl] = jnp.where(m, 1.0, 0.0).astype(jnp.float32)
                so_v[r, sl] = jnp.where(m, _REPLACE_ID, seq_v[r, sl])
                out.append(carry[r] + jnp.where(m, lv, 0.0))
            return tuple(out)
        return accs

    d_pc.wait()
    d_s0.wait()
    d_u0.wait()
    accs0 = run_pair(seq0_v, u0_v, so0_v, mk0_v)
    o_s0 = cp(so0_v, seq_out_hbm.at[pl.ds(base, _PAIR)], sem_o)
    o_m0 = cp(mk0_v, mask_hbm.at[pl.ds(base, _PAIR)], sem_o)

    d_s1.wait()
    d_u1.wait()
    accs1 = run_pair(seq1_v, u1_v, so1_v, mk1_v)
    o_s1 = cp(so1_v, seq_out_hbm.at[pl.ds(base + _PAIR, _PAIR)], sem_o)
    o_m1 = cp(mk1_v, mask_hbm.at[pl.ds(base + _PAIR, _PAIR)], sem_o)

    d_c.wait()
    cvec = c_v[:]
    for r in range(_PAIR):
        logits_v[r, :] = jnp.broadcast_to(jnp.sum(accs0[r]), (_LANES,)) - cvec
        logits_v[_PAIR + r, :] = (
            jnp.broadcast_to(jnp.sum(accs1[r]), (_LANES,)) - cvec)

    pltpu.sync_copy(logits_v, logits_hbm.at[pl.ds(base, _ROWS_PER_W)])

    o_s0.wait()
    o_m0.wait()
    o_s1.wait()
    o_m1.wait()


@jax.jit
def kernel(sequence, prob_mask_logits, u):
    B, L = sequence.shape

    pc3, c = pl.pallas_call(
        _prep_body,
        out_shape=(
            jax.ShapeDtypeStruct((2, _LANES, L // _LANES), jnp.float32),
            jax.ShapeDtypeStruct((_LANES,), jnp.float32),
        ),
    )(prob_mask_logits.reshape(_LANES, L // _LANES))
    pc = pc3.reshape(2, L)

    mesh = plsc.VectorSubcoreMesh(
        core_axis_name="c", subcore_axis_name="s",
        num_cores=_NC, num_subcores=_NS)

    sc = pl.kernel(
        _sc_body,
        out_type=(
            jax.ShapeDtypeStruct((B, L), jnp.int32),
            jax.ShapeDtypeStruct((B, L), jnp.float32),
            jax.ShapeDtypeStruct((B, _LANES), jnp.float32),
        ),
        mesh=mesh,
        compiler_params=pltpu.CompilerParams(needs_layout_passes=False),
        scratch_types=[
            pltpu.VMEM((2, L), jnp.float32),
            pltpu.VMEM((_LANES,), jnp.float32),
            pltpu.VMEM((_PAIR, L), jnp.int32),
            pltpu.VMEM((_PAIR, L), jnp.float32),
            pltpu.VMEM((_PAIR, L), jnp.int32),
            pltpu.VMEM((_PAIR, L), jnp.float32),
            pltpu.VMEM((_PAIR, L), jnp.int32),
            pltpu.VMEM((_PAIR, L), jnp.float32),
            pltpu.VMEM((_PAIR, L), jnp.int32),
            pltpu.VMEM((_PAIR, L), jnp.float32),
            pltpu.VMEM((_ROWS_PER_W, _LANES), jnp.float32),
            pltpu.SemaphoreType.DMA,
            pltpu.SemaphoreType.DMA,
            pltpu.SemaphoreType.DMA,
            pltpu.SemaphoreType.DMA,
        ],
    )
    seq_out, hard_mask, logits_pad = sc(sequence, u, pc, c)
    return (seq_out, logits_pad[:, 0], hard_mask)


# speedup vs baseline: 1.0335x; 1.0014x over previous
import functools

import jax
import jax.numpy as jnp
from jax import lax
from jax.experimental import pallas as pl
from jax.experimental.pallas import tpu as pltpu
from jax.experimental.pallas import tpu_sc as plsc

_REPLACE_ID = 100001

_B = 128
_L = 2048
_LANES = 16
_NC = 2
_NS = 16
_NW = _NC * _NS
_ROWS_PER_W = _B // _NW
_PAIR = 2
_BANKS = 8


def _prep_body(pml_ref, pc_ref, c_ref):
    x = pml_ref[...]
    pc_ref[0] = 1.0 / (1.0 + jnp.exp(-x))
    pc_ref[1] = x
    sp = jnp.maximum(x, 0.0) + jnp.log1p(jnp.exp(-jnp.abs(x)))
    c_ref[...] = jnp.broadcast_to(jnp.sum(sp), (_LANES,))


def _sc_body(seq_hbm, u_hbm, pc_hbm, c_hbm,
             seq_out_hbm, mask_hbm, logits_hbm,
             pc_v, c_v,
             seq0_v, u0_v, so0_v, mk0_v,
             seq1_v, u1_v, so1_v, mk1_v,
             logits_v,
             sem_m, sem_0, sem_1, sem_o):
    cid = lax.axis_index("c")
    sid = lax.axis_index("s")
    wid = cid * _NS + sid
    base = wid * _ROWS_PER_W

    cp = pltpu.async_copy
    d_pc = cp(pc_hbm, pc_v, sem_m)
    d_c = cp(c_hbm, c_v, sem_m)
    d_s0 = cp(seq_hbm.at[pl.ds(base, _PAIR)], seq0_v, sem_0)
    d_u0 = cp(u_hbm.at[pl.ds(base, _PAIR)], u0_v, sem_0)
    d_s1 = cp(seq_hbm.at[pl.ds(base + _PAIR, _PAIR)], seq1_v, sem_1)
    d_u1 = cp(u_hbm.at[pl.ds(base + _PAIR, _PAIR)], u1_v, sem_1)

    zero = jnp.zeros((_LANES,), jnp.float32)

    def run_pair(seq_v, u_v, so_v, mk_v):
        @plsc.parallel_loop(0, _L, step=_LANES, unroll=2,
                            carry=(zero,) * _PAIR)
        def accs(off, carry):
            sl = pl.ds(off, _LANES)
            pv = pc_v[0, sl]
            lv = pc_v[1, sl]
            out = []
            for r in range(_PAIR):
                m = u_v[r, sl] < pv
                mk_v[r, sl] = jnp.where(m, 1.0, 0.0).astype(jnp.float32)
                so_v[r, sl] = jnp.where(m, _REPLACE_ID, seq_v[r, sl])
                out.append(carry[r] + jnp.where(m, lv, 0.0))
            return tuple(out)
        return accs

    d_pc.wait()
    d_s0.wait()
    d_u0.wait()
    accs0 = run_pair(seq0_v, u0_v, so0_v, mk0_v)
    o_s0 = cp(so0_v, seq_out_hbm.at[pl.ds(base, _PAIR)], sem_o)
    o_m0 = cp(mk0_v, mask_hbm.at[pl.ds(base, _PAIR)], sem_o)

    d_s1.wait()
    d_u1.wait()
    accs1 = run_pair(seq1_v, u1_v, so1_v, mk1_v)
    o_s1 = cp(so1_v, seq_out_hbm.at[pl.ds(base + _PAIR, _PAIR)], sem_o)
    o_m1 = cp(mk1_v, mask_hbm.at[pl.ds(base + _PAIR, _PAIR)], sem_o)

    d_c.wait()
    cvec = c_v[:]
    for r in range(_PAIR):
        logits_v[r, :] = jnp.broadcast_to(jnp.sum(accs0[r]), (_LANES,)) - cvec
        logits_v[_PAIR + r, :] = (
            jnp.broadcast_to(jnp.sum(accs1[r]), (_LANES,)) - cvec)

    pltpu.sync_copy(logits_v, logits_hbm.at[pl.ds(base, _ROWS_PER_W)])

    o_s0.wait()
    o_m0.wait()
    o_s1.wait()
    o_m1.wait()


@jax.jit
def kernel(sequence, prob_mask_logits, u):
    B, L = sequence.shape

    pc3, c = pl.pallas_call(
        _prep_body,
        out_shape=(
            jax.ShapeDtypeStruct((2, _LANES, L // _LANES), jnp.float32),
            jax.ShapeDtypeStruct((_LANES,), jnp.float32),
        ),
    )(prob_mask_logits.reshape(_LANES, L // _LANES))
    pc = pc3.reshape(2, L)

    mesh = plsc.VectorSubcoreMesh(
        core_axis_name="c", subcore_axis_name="s",
        num_cores=_NC, num_subcores=_NS)

    sc = pl.kernel(
        _sc_body,
        out_type=(
            jax.ShapeDtypeStruct((B, L), jnp.int32),
            jax.ShapeDtypeStruct((B, L), jnp.float32),
            jax.ShapeDtypeStruct((B, _LANES), jnp.float32),
        ),
        mesh=mesh,
        compiler_params=pltpu.CompilerParams(needs_layout_passes=False),
        scratch_types=[
            pltpu.VMEM((2, L), jnp.float32),
            pltpu.VMEM((_LANES,), jnp.float32),
            pltpu.VMEM((_PAIR, L), jnp.int32),
            pltpu.VMEM((_PAIR, L), jnp.float32),
            pltpu.VMEM((_PAIR, L), jnp.int32),
            pltpu.VMEM((_PAIR, L), jnp.float32),
            pltpu.VMEM((_PAIR, L), jnp.int32),
            pltpu.VMEM((_PAIR, L), jnp.float32),
            pltpu.VMEM((_PAIR, L), jnp.int32),
            pltpu.VMEM((_PAIR, L), jnp.float32),
            pltpu.VMEM((_ROWS_PER_W, _LANES), jnp.float32),
            pltpu.SemaphoreType.DMA,
            pltpu.SemaphoreType.DMA,
            pltpu.SemaphoreType.DMA,
            pltpu.SemaphoreType.DMA,
        ],
    )
    seq_out, hard_mask, logits_pad = sc(sequence, u, pc, c)
    return (seq_out, logits_pad[:, 0], hard_mask)


# speedup vs baseline: 1.0450x; 1.0111x over previous
import functools

import jax
import jax.numpy as jnp
from jax import lax
from jax.experimental import pallas as pl
from jax.experimental.pallas import tpu as pltpu
from jax.experimental.pallas import tpu_sc as plsc

_REPLACE_ID = 100001

_B = 128
_L = 2048
_LANES = 16
_NC = 2
_NS = 16
_NW = _NC * _NS
_ROWS_PER_W = _B // _NW
_PAIR = 2
_BANKS = 8


def _prep_body(pml_ref, p_ref, c_ref):
    x = pml_ref[...]
    p_ref[...] = 1.0 / (1.0 + jnp.exp(-x))
    sp = jnp.maximum(x, 0.0) + jnp.log1p(jnp.exp(-jnp.abs(x)))
    c_ref[...] = jnp.broadcast_to(jnp.sum(sp), (_LANES,))


def _sc_body(seq_hbm, u_hbm, p_hbm, pml_hbm, c_hbm,
             seq_out_hbm, mask_hbm, logits_hbm,
             p_v, pml_v, c_v,
             seq0_v, u0_v, so0_v, mk0_v,
             seq1_v, u1_v, so1_v, mk1_v,
             logits_v,
             sem_m, sem_0, sem_1, sem_o):
    cid = lax.axis_index("c")
    sid = lax.axis_index("s")
    wid = cid * _NS + sid
    base = wid * _ROWS_PER_W

    cp = pltpu.async_copy
    d_p = cp(p_hbm, p_v, sem_m)
    d_l = cp(pml_hbm, pml_v, sem_m)
    d_c = cp(c_hbm, c_v, sem_m)
    d_s0 = cp(seq_hbm.at[pl.ds(base, _PAIR)], seq0_v, sem_0)
    d_u0 = cp(u_hbm.at[pl.ds(base, _PAIR)], u0_v, sem_0)
    d_s1 = cp(seq_hbm.at[pl.ds(base + _PAIR, _PAIR)], seq1_v, sem_1)
    d_u1 = cp(u_hbm.at[pl.ds(base + _PAIR, _PAIR)], u1_v, sem_1)

    zero = jnp.zeros((_LANES,), jnp.float32)

    def run_pair(seq_v, u_v, so_v, mk_v):
        @plsc.parallel_loop(0, _L, step=_LANES, unroll=2,
                            carry=(zero,) * _PAIR)
        def accs(off, carry):
            sl = pl.ds(off, _LANES)
            pv = p_v[sl]
            lv = pml_v[sl]
            out = []
            for r in range(_PAIR):
                m = u_v[r, sl] < pv
                mk_v[r, sl] = jnp.where(m, 1.0, 0.0).astype(jnp.float32)
                so_v[r, sl] = jnp.where(m, _REPLACE_ID, seq_v[r, sl])
                out.append(carry[r] + jnp.where(m, lv, 0.0))
            return tuple(out)
        return accs

    d_p.wait()
    d_l.wait()
    d_s0.wait()
    d_u0.wait()
    accs0 = run_pair(seq0_v, u0_v, so0_v, mk0_v)
    o_s0 = cp(so0_v, seq_out_hbm.at[pl.ds(base, _PAIR)], sem_o)
    o_m0 = cp(mk0_v, mask_hbm.at[pl.ds(base, _PAIR)], sem_o)

    d_s1.wait()
    d_u1.wait()
    accs1 = run_pair(seq1_v, u1_v, so1_v, mk1_v)
    o_s1 = cp(so1_v, seq_out_hbm.at[pl.ds(base + _PAIR, _PAIR)], sem_o)
    o_m1 = cp(mk1_v, mask_hbm.at[pl.ds(base + _PAIR, _PAIR)], sem_o)

    d_c.wait()
    cvec = c_v[:]
    for r in range(_PAIR):
        logits_v[r, :] = jnp.broadcast_to(jnp.sum(accs0[r]), (_LANES,)) - cvec
        logits_v[_PAIR + r, :] = (
            jnp.broadcast_to(jnp.sum(accs1[r]), (_LANES,)) - cvec)

    pltpu.sync_copy(logits_v, logits_hbm.at[pl.ds(base, _ROWS_PER_W)])

    o_s0.wait()
    o_m0.wait()
    o_s1.wait()
    o_m1.wait()


@jax.jit
def kernel(sequence, prob_mask_logits, u):
    B, L = sequence.shape

    p2, c = pl.pallas_call(
        _prep_body,
        out_shape=(
            jax.ShapeDtypeStruct((_LANES, L // _LANES), jnp.float32),
            jax.ShapeDtypeStruct((_LANES,), jnp.float32),
        ),
    )(prob_mask_logits.reshape(_LANES, L // _LANES))
    p = p2.reshape(L)

    mesh = plsc.VectorSubcoreMesh(
        core_axis_name="c", subcore_axis_name="s",
        num_cores=_NC, num_subcores=_NS)

    sc = pl.kernel(
        _sc_body,
        out_type=(
            jax.ShapeDtypeStruct((B, L), jnp.int32),
            jax.ShapeDtypeStruct((B, L), jnp.float32),
            jax.ShapeDtypeStruct((B, _LANES), jnp.float32),
        ),
        mesh=mesh,
        compiler_params=pltpu.CompilerParams(needs_layout_passes=False),
        scratch_types=[
            pltpu.VMEM((L,), jnp.float32),
            pltpu.VMEM((L,), jnp.float32),
            pltpu.VMEM((_LANES,), jnp.float32),
            pltpu.VMEM((_PAIR, L), jnp.int32),
            pltpu.VMEM((_PAIR, L), jnp.float32),
            pltpu.VMEM((_PAIR, L), jnp.int32),
            pltpu.VMEM((_PAIR, L), jnp.float32),
            pltpu.VMEM((_PAIR, L), jnp.int32),
            pltpu.VMEM((_PAIR, L), jnp.float32),
            pltpu.VMEM((_PAIR, L), jnp.int32),
            pltpu.VMEM((_PAIR, L), jnp.float32),
            pltpu.VMEM((_ROWS_PER_W, _LANES), jnp.float32),
            pltpu.SemaphoreType.DMA,
            pltpu.SemaphoreType.DMA,
            pltpu.SemaphoreType.DMA,
            pltpu.SemaphoreType.DMA,
        ],
    )
    seq_out, hard_mask, logits_pad = sc(sequence, u, p, prob_mask_logits, c)
    return (seq_out, logits_pad[:, 0], hard_mask)
